# Initial kernel scaffold; baseline (speedup 1.0000x reference)
import jax
import jax.numpy as jnp
from jax.experimental import pallas as pl
from jax.experimental.pallas import tpu as pltpu

_BEAM = 4
_VOCAB = 100000
_PAD = 0
_SOS = 1
_EOS = 2
_ROWS = 8
_NEG_INF = jnp.float32(-jnp.inf)
_BIG_I32 = jnp.int32(2147483647)


def _step(x_ref, sc_ref, eos_ref, ts_ref, wi_ref, bi_ref):
    x = x_ref[...]
    sc = sc_ref[...]
    eosb = eos_ref[...] != 0

    m = jnp.max(x, axis=1, keepdims=True)
    s = jnp.sum(jnp.exp(x - m), axis=1, keepdims=True)
    c = sc - m - jnp.log(s)

    col = jax.lax.broadcasted_iota(jnp.int32, (_ROWS, _VOCAB), 1)
    adj = x + c
    adj = jnp.where(eosb, _NEG_INF, adj)
    adj = jnp.where(col < 2, _NEG_INF, adj)
    adj = jnp.where((col == _EOS) & eosb, sc, adj)

    row = jax.lax.broadcasted_iota(jnp.int32, (_ROWS, _VOCAB), 0)

    kcol = jax.lax.broadcasted_iota(jnp.int32, (1, 2, _BEAM), 2)
    brow = jax.lax.broadcasted_iota(jnp.int32, (1, 2, _BEAM), 1)
    ts = jnp.zeros((1, 2, _BEAM), jnp.float32)
    wi = jnp.zeros((1, 2, _BEAM), jnp.int32)
    bi = jnp.zeros((1, 2, _BEAM), jnp.int32)

    for b in range(2):
        ab = adj[b * _BEAM:(b + 1) * _BEAM]
        fl = (row[b * _BEAM:(b + 1) * _BEAM] - b * _BEAM) * _VOCAB + \
            col[b * _BEAM:(b + 1) * _BEAM]
        for k in range(_BEAM):
            mk = jnp.max(ab)
            idx = jnp.min(jnp.where(ab == mk, fl, _BIG_I32))
            ab = jnp.where(fl == idx, _NEG_INF, ab)
            beam = idx // _VOCAB
            word = idx - beam * _VOCAB
            hit = (brow == b) & (kcol == k)
            ts = jnp.where(hit, mk, ts)
            wi = jnp.where(hit, word, wi)
            bi = jnp.where(hit, beam, bi)

    ts_ref[...] = ts
    wi_ref[...] = wi
    bi_ref[...] = bi


def kernel(logits, scores, generated_tokens, position):
    n_rows = logits.shape[0]
    n_batch = n_rows // _BEAM
    sc_sel = jnp.take(scores, position - 1, axis=2).reshape(n_rows, 1)
    eos = (jnp.take(generated_tokens, position, axis=1) == _EOS)
    eos = eos.astype(jnp.int32).reshape(n_rows, 1)

    grid = (n_rows // _ROWS,)
    out3 = (grid[0], 2, _BEAM)
    ts, wi, bi = pl.pallas_call(
        _step,
        grid=grid,
        in_specs=[
            pl.BlockSpec((_ROWS, _VOCAB), lambda i: (i, 0)),
            pl.BlockSpec((_ROWS, 1), lambda i: (i, 0)),
            pl.BlockSpec((_ROWS, 1), lambda i: (i, 0)),
        ],
        out_specs=[
            pl.BlockSpec((1, 2, _BEAM), lambda i: (i, 0, 0)),
            pl.BlockSpec((1, 2, _BEAM), lambda i: (i, 0, 0)),
            pl.BlockSpec((1, 2, _BEAM), lambda i: (i, 0, 0)),
        ],
        out_shape=[
            jax.ShapeDtypeStruct(out3, jnp.float32),
            jax.ShapeDtypeStruct(out3, jnp.int32),
            jax.ShapeDtypeStruct(out3, jnp.int32),
        ],
        compiler_params=pltpu.CompilerParams(
            dimension_semantics=("parallel",),
        ),
    )(logits, sc_sel, eos)
    return (ts.reshape(n_batch, _BEAM),
            wi.reshape(n_batch, _BEAM),
            bi.reshape(n_batch, _BEAM))


# speedup vs baseline: 1.5536x; 1.5536x over previous
import jax
import jax.numpy as jnp
from jax.experimental import pallas as pl
from jax.experimental.pallas import tpu as pltpu

_BEAM = 4
_VOCAB = 100000
_PAD = 0
_SOS = 1
_EOS = 2
_ROWS = 8
_NEG_INF = float("-inf")
_BIG_I32 = 2147483647


def _step(x_ref, sc_ref, eos_ref, ts_ref, wi_ref, bi_ref):
    x = x_ref[...]
    sc = sc_ref[...]
    eosb = eos_ref[...] != 0

    m = jnp.max(x, axis=1, keepdims=True)
    s = jnp.sum(jnp.exp(x - m), axis=1, keepdims=True)
    c = sc - m - jnp.log(s)

    col = jax.lax.broadcasted_iota(jnp.int32, (_ROWS, _VOCAB), 1)
    adj = x + c
    adj = jnp.where(eosb, _NEG_INF, adj)
    adj = jnp.where(col < 2, _NEG_INF, adj)
    adj = jnp.where((col == _EOS) & eosb, sc, adj)

    row = jax.lax.broadcasted_iota(jnp.int32, (_ROWS, _VOCAB), 0)

    kcol = jax.lax.broadcasted_iota(jnp.int32, (1, 2, _BEAM), 2)
    brow = jax.lax.broadcasted_iota(jnp.int32, (1, 2, _BEAM), 1)
    ts = jnp.zeros((1, 2, _BEAM), jnp.float32)
    wi = jnp.zeros((1, 2, _BEAM), jnp.int32)
    bi = jnp.zeros((1, 2, _BEAM), jnp.int32)

    for b in range(2):
        ab = adj[b * _BEAM:(b + 1) * _BEAM]
        fl = (row[b * _BEAM:(b + 1) * _BEAM] - b * _BEAM) * _VOCAB + \
            col[b * _BEAM:(b + 1) * _BEAM]
        for k in range(_BEAM):
            mk = jnp.max(ab)
            idx = jnp.min(jnp.where(ab == mk, fl, _BIG_I32))
            ab = jnp.where(fl == idx, _NEG_INF, ab)
            beam = idx // _VOCAB
            word = idx - beam * _VOCAB
            hit = (brow == b) & (kcol == k)
            ts = jnp.where(hit, mk, ts)
            wi = jnp.where(hit, word, wi)
            bi = jnp.where(hit, beam, bi)

    ts_ref[...] = ts
    wi_ref[...] = wi
    bi_ref[...] = bi


def kernel(logits, scores, generated_tokens, position):
    n_rows = logits.shape[0]
    n_batch = n_rows // _BEAM
    sc_sel = jnp.take(scores, position - 1, axis=2).reshape(n_rows, 1)
    eos = (jnp.take(generated_tokens, position, axis=1) == _EOS)
    eos = eos.astype(jnp.int32).reshape(n_rows, 1)

    grid = (n_rows // _ROWS,)
    out3 = (grid[0], 2, _BEAM)
    ts, wi, bi = pl.pallas_call(
        _step,
        grid=grid,
        in_specs=[
            pl.BlockSpec((_ROWS, _VOCAB), lambda i: (i, 0)),
            pl.BlockSpec((_ROWS, 1), lambda i: (i, 0)),
            pl.BlockSpec((_ROWS, 1), lambda i: (i, 0)),
        ],
        out_specs=[
            pl.BlockSpec((1, 2, _BEAM), lambda i: (i, 0, 0)),
            pl.BlockSpec((1, 2, _BEAM), lambda i: (i, 0, 0)),
            pl.BlockSpec((1, 2, _BEAM), lambda i: (i, 0, 0)),
        ],
        out_shape=[
            jax.ShapeDtypeStruct(out3, jnp.float32),
            jax.ShapeDtypeStruct(out3, jnp.int32),
            jax.ShapeDtypeStruct(out3, jnp.int32),
        ],
        compiler_params=pltpu.CompilerParams(
            dimension_semantics=("parallel",),
        ),
    )(logits, sc_sel, eos)
    return (ts.reshape(n_batch, _BEAM),
            wi.reshape(n_batch, _BEAM),
            bi.reshape(n_batch, _BEAM))


# speedup vs baseline: 2.3713x; 1.5263x over previous
import jax
import jax.numpy as jnp
from jax.experimental import pallas as pl
from jax.experimental.pallas import tpu as pltpu

_BEAM = 4
_VOCAB = 100000
_EOS = 2
_ROWS = 8
_LANE = 128
_NFULL = _VOCAB // _LANE
_TAILSTART = _VOCAB - _LANE
_TAILSKIP = _NFULL * _LANE - _TAILSTART
_NEG = float("-inf")
_BIG = 2147483647


def _insert(state, v, off):
    t1, t2, t3, t4, i1, i2, i3, i4 = state
    b1 = v > t1
    b2 = v > t2
    b3 = v > t3
    b4 = v > t4
    n1 = jnp.where(b1, v, t1)
    n2 = jnp.where(b1, t1, jnp.where(b2, v, t2))
    n3 = jnp.where(b2, t2, jnp.where(b3, v, t3))
    n4 = jnp.where(b3, t3, jnp.where(b4, v, t4))
    j1 = jnp.where(b1, off, i1)
    j2 = jnp.where(b1, i1, jnp.where(b2, off, i2))
    j3 = jnp.where(b2, i2, jnp.where(b3, off, i3))
    j4 = jnp.where(b3, i3, jnp.where(b4, off, i4))
    return (n1, n2, n3, n4, j1, j2, j3, j4)


def _step(x_ref, sc_ref, eos_ref, ts_ref, wi_ref, bi_ref):
    sc = sc_ref[...]
    eosb = eos_ref[...] != 0

    lane = jax.lax.broadcasted_iota(jnp.int32, (_ROWS, _LANE), 1)
    row8 = jax.lax.broadcasted_iota(jnp.int32, (_ROWS, _LANE), 0)
    row4 = row8 % _BEAM

    v0 = x_ref[:, 0:_LANE]
    mrow = v0
    neg = jnp.full((_ROWS, _LANE), _NEG, jnp.float32)
    zero = jnp.zeros((_ROWS, _LANE), jnp.int32)
    state = (jnp.where(lane < 2, _NEG, v0), neg, neg, neg,
             zero, zero, zero, zero)

    def body(c, carry):
        mrow, state = carry
        off = c * _LANE
        v = x_ref[:, pl.ds(pl.multiple_of(off, _LANE), _LANE)]
        return (jnp.maximum(mrow, v), _insert(state, v, off))

    mrow, state = jax.lax.fori_loop(1, _NFULL, body, (mrow, state))

    vt = x_ref[:, _TAILSTART:_TAILSTART + _LANE]
    mrow = jnp.maximum(mrow, jnp.where(lane >= _TAILSKIP, vt, _NEG))
    state = _insert(state, jnp.where(lane >= _TAILSKIP, vt, _NEG), _TAILSTART)
    t1, t2, t3, t4, i1, i2, i3, i4 = state

    m = jnp.max(mrow, axis=1, keepdims=True)
    x = x_ref[...]
    s = jnp.sum(jnp.exp(x - m), axis=1, keepdims=True)
    crow = sc - m - jnp.log(s)
    crow = jnp.where(eosb, _NEG, crow)

    cv = [t1 + crow, t2 + crow, t3 + crow, t4 + crow,
          jnp.where(eosb & (lane == _EOS), sc, _NEG)]
    cf = [row4 * _VOCAB + i1 + lane, row4 * _VOCAB + i2 + lane,
          row4 * _VOCAB + i3 + lane, row4 * _VOCAB + i4 + lane,
          row4 * _VOCAB + lane]

    kcol = jax.lax.broadcasted_iota(jnp.int32, (1, 2, _BEAM), 2)
    brow = jax.lax.broadcasted_iota(jnp.int32, (1, 2, _BEAM), 1)
    ts = jnp.zeros((1, 2, _BEAM), jnp.float32)
    wi = jnp.zeros((1, 2, _BEAM), jnp.int32)
    bi = jnp.zeros((1, 2, _BEAM), jnp.int32)

    for b in range(2):
        selb = (row8 // _BEAM) == b
        av = [jnp.where(selb, v, _NEG) for v in cv]
        for k in range(_BEAM):
            mk = jnp.max(jnp.maximum(
                jnp.maximum(jnp.maximum(av[0], av[1]),
                            jnp.maximum(av[2], av[3])), av[4]))
            cand_i = [jnp.where(v == mk, f, _BIG) for v, f in zip(av, cf)]
            idx = jnp.min(jnp.minimum(
                jnp.minimum(jnp.minimum(cand_i[0], cand_i[1]),
                            jnp.minimum(cand_i[2], cand_i[3])), cand_i[4]))
            av = [jnp.where((f == idx) & (v == mk), _NEG, v)
                  for v, f in zip(av, cf)]
            beam = idx // _VOCAB
            word = idx - beam * _VOCAB
            hit = (brow == b) & (kcol == k)
            ts = jnp.where(hit, mk, ts)
            wi = jnp.where(hit, word, wi)
            bi = jnp.where(hit, beam, bi)

    ts_ref[...] = ts
    wi_ref[...] = wi
    bi_ref[...] = bi


def kernel(logits, scores, generated_tokens, position):
    n_rows = logits.shape[0]
    n_batch = n_rows // _BEAM
    sc_sel = jnp.take(scores, position - 1, axis=2).reshape(n_rows, 1)
    eos = (jnp.take(generated_tokens, position, axis=1) == _EOS)
    eos = eos.astype(jnp.int32).reshape(n_rows, 1)

    grid = (n_rows // _ROWS,)
    out3 = (grid[0], 2, _BEAM)
    ts, wi, bi = pl.pallas_call(
        _step,
        grid=grid,
        in_specs=[
            pl.BlockSpec((_ROWS, _VOCAB), lambda i: (i, 0)),
            pl.BlockSpec((_ROWS, 1), lambda i: (i, 0)),
            pl.BlockSpec((_ROWS, 1), lambda i: (i, 0)),
        ],
        out_specs=[
            pl.BlockSpec((1, 2, _BEAM), lambda i: (i, 0, 0)),
            pl.BlockSpec((1, 2, _BEAM), lambda i: (i, 0, 0)),
            pl.BlockSpec((1, 2, _BEAM), lambda i: (i, 0, 0)),
        ],
        out_shape=[
            jax.ShapeDtypeStruct(out3, jnp.float32),
            jax.ShapeDtypeStruct(out3, jnp.int32),
            jax.ShapeDtypeStruct(out3, jnp.int32),
        ],
        compiler_params=pltpu.CompilerParams(
            dimension_semantics=("parallel",),
        ),
    )(logits, sc_sel, eos)
    return (ts.reshape(n_batch, _BEAM),
            wi.reshape(n_batch, _BEAM),
            bi.reshape(n_batch, _BEAM))


# speedup vs baseline: 2.6252x; 1.1071x over previous
import jax
import jax.numpy as jnp
from jax.experimental import pallas as pl
from jax.experimental.pallas import tpu as pltpu

_BEAM = 4
_VOCAB = 100000
_EOS = 2
_ROWS = 8
_LANE = 128
_NFULL = _VOCAB // _LANE
_TAILSTART = _VOCAB - _LANE
_TAILSKIP = _NFULL * _LANE - _TAILSTART
_PHASES = 4
_NITER = (_NFULL - 1) // _PHASES
_NEG = float("-inf")
_BIG = 2147483647


def _insert(state, v, off):
    t1, t2, t3, t4, i1, i2, i3, i4 = state
    b1 = v > t1
    b2 = v > t2
    b3 = v > t3
    b4 = v > t4
    n1 = jnp.where(b1, v, t1)
    n2 = jnp.where(b1, t1, jnp.where(b2, v, t2))
    n3 = jnp.where(b2, t2, jnp.where(b3, v, t3))
    n4 = jnp.where(b3, t3, jnp.where(b4, v, t4))
    j1 = jnp.where(b1, off, i1)
    j2 = jnp.where(b1, i1, jnp.where(b2, off, i2))
    j3 = jnp.where(b2, i2, jnp.where(b3, off, i3))
    j4 = jnp.where(b3, i3, jnp.where(b4, off, i4))
    return (n1, n2, n3, n4, j1, j2, j3, j4)


def _step(x_ref, sc_ref, eos_ref, ts_ref, wi_ref, bi_ref):
    sc = sc_ref[...]
    eosb = eos_ref[...] != 0

    lane = jax.lax.broadcasted_iota(jnp.int32, (_ROWS, _LANE), 1)
    row8 = jax.lax.broadcasted_iota(jnp.int32, (_ROWS, _LANE), 0)
    row4 = row8 % _BEAM

    v0 = x_ref[:, 0:_LANE]
    neg = jnp.full((_ROWS, _LANE), _NEG, jnp.float32)
    zero = jnp.zeros((_ROWS, _LANE), jnp.int32)
    empty = (neg, neg, neg, neg, zero, zero, zero, zero)
    states = [(jnp.where(lane < 2, _NEG, v0), neg, neg, neg,
               zero, zero, zero, zero)] + [empty] * (_PHASES - 1)
    mrows = [v0] + [neg] * (_PHASES - 1)

    def body(j, carry):
        mrows, states = carry
        mrows = list(mrows)
        states = list(states)
        base = (1 + _PHASES * j) * _LANE
        for p in range(_PHASES):
            off = base + p * _LANE
            v = x_ref[:, pl.ds(pl.multiple_of(off, _LANE), _LANE)]
            mrows[p] = jnp.maximum(mrows[p], v)
            states[p] = _insert(states[p], v, off)
        return (tuple(mrows), tuple(states))

    mrows, states = jax.lax.fori_loop(
        0, _NITER, body, (tuple(mrows), tuple(states)))
    mrows = list(mrows)
    states = list(states)

    vt = jnp.where(lane >= _TAILSKIP, x_ref[:, _TAILSTART:_TAILSTART + _LANE],
                   _NEG)
    mrows[0] = jnp.maximum(mrows[0], vt)
    states[0] = _insert(states[0], vt, _TAILSTART)

    mrow = jnp.maximum(jnp.maximum(mrows[0], mrows[1]),
                       jnp.maximum(mrows[2], mrows[3]))
    m = jnp.max(mrow, axis=1, keepdims=True)
    x = x_ref[...]
    s = jnp.sum(jnp.exp(x - m), axis=1, keepdims=True)
    crow = sc - m - jnp.log(s)
    crow = jnp.where(eosb, _NEG, crow)

    cv = []
    cf = []
    for st in states:
        t1, t2, t3, t4, i1, i2, i3, i4 = st
        cv += [t1 + crow, t2 + crow, t3 + crow, t4 + crow]
        cf += [row4 * _VOCAB + i1 + lane, row4 * _VOCAB + i2 + lane,
               row4 * _VOCAB + i3 + lane, row4 * _VOCAB + i4 + lane]
    cv.append(jnp.where(eosb & (lane == _EOS), sc, _NEG))
    cf.append(row4 * _VOCAB + lane)

    def _treemax(xs):
        while len(xs) > 1:
            xs = [jnp.maximum(a, b) for a, b in zip(xs[::2], xs[1::2])] + \
                ([xs[-1]] if len(xs) % 2 else [])
        return xs[0]

    def _treemin(xs):
        while len(xs) > 1:
            xs = [jnp.minimum(a, b) for a, b in zip(xs[::2], xs[1::2])] + \
                ([xs[-1]] if len(xs) % 2 else [])
        return xs[0]

    kcol = jax.lax.broadcasted_iota(jnp.int32, (1, 2, _BEAM), 2)
    brow = jax.lax.broadcasted_iota(jnp.int32, (1, 2, _BEAM), 1)
    ts = jnp.zeros((1, 2, _BEAM), jnp.float32)
    wi = jnp.zeros((1, 2, _BEAM), jnp.int32)
    bi = jnp.zeros((1, 2, _BEAM), jnp.int32)

    for b in range(2):
        selb = (row8 // _BEAM) == b
        av = [jnp.where(selb, v, _NEG) for v in cv]
        for k in range(_BEAM):
            mk = jnp.max(_treemax(av))
            idx = jnp.min(_treemin(
                [jnp.where(v == mk, f, _BIG) for v, f in zip(av, cf)]))
            av = [jnp.where((f == idx) & (v == mk), _NEG, v)
                  for v, f in zip(av, cf)]
            beam = idx // _VOCAB
            word = idx - beam * _VOCAB
            hit = (brow == b) & (kcol == k)
            ts = jnp.where(hit, mk, ts)
            wi = jnp.where(hit, word, wi)
            bi = jnp.where(hit, beam, bi)

    ts_ref[...] = ts
    wi_ref[...] = wi
    bi_ref[...] = bi


def kernel(logits, scores, generated_tokens, position):
    n_rows = logits.shape[0]
    n_batch = n_rows // _BEAM
    sc_sel = jnp.take(scores, position - 1, axis=2).reshape(n_rows, 1)
    eos = (jnp.take(generated_tokens, position, axis=1) == _EOS)
    eos = eos.astype(jnp.int32).reshape(n_rows, 1)

    grid = (n_rows // _ROWS,)
    out3 = (grid[0], 2, _BEAM)
    ts, wi, bi = pl.pallas_call(
        _step,
        grid=grid,
        in_specs=[
            pl.BlockSpec((_ROWS, _VOCAB), lambda i: (i, 0)),
            pl.BlockSpec((_ROWS, 1), lambda i: (i, 0)),
            pl.BlockSpec((_ROWS, 1), lambda i: (i, 0)),
        ],
        out_specs=[
            pl.BlockSpec((1, 2, _BEAM), lambda i: (i, 0, 0)),
            pl.BlockSpec((1, 2, _BEAM), lambda i: (i, 0, 0)),
            pl.BlockSpec((1, 2, _BEAM), lambda i: (i, 0, 0)),
        ],
        out_shape=[
            jax.ShapeDtypeStruct(out3, jnp.float32),
            jax.ShapeDtypeStruct(out3, jnp.int32),
            jax.ShapeDtypeStruct(out3, jnp.int32),
        ],
        compiler_params=pltpu.CompilerParams(
            dimension_semantics=("parallel",),
        ),
    )(logits, sc_sel, eos)
    return (ts.reshape(n_batch, _BEAM),
            wi.reshape(n_batch, _BEAM),
            bi.reshape(n_batch, _BEAM))


# speedup vs baseline: 3.2761x; 1.2479x over previous
import jax
import jax.numpy as jnp
from jax.experimental import pallas as pl
from jax.experimental.pallas import tpu as pltpu

_BEAM = 4
_VOCAB = 100000
_EOS = 2
_ROWS = 8
_LANE = 128
_NFULL = _VOCAB // _LANE
_TAILSTART = _VOCAB - _LANE
_TAILSKIP = _NFULL * _LANE - _TAILSTART
_PHASES = 4
_NITER = (_NFULL - 1) // _PHASES
_NEG = float("-inf")
_BIG = 2147483647


def _insert(state, v, off):
    t1, t2, t3, t4, i1, i2, i3, i4 = state
    b1 = v > t1
    b2 = v > t2
    b3 = v > t3
    b4 = v > t4
    n1 = jnp.where(b1, v, t1)
    n2 = jnp.where(b1, t1, jnp.where(b2, v, t2))
    n3 = jnp.where(b2, t2, jnp.where(b3, v, t3))
    n4 = jnp.where(b3, t3, jnp.where(b4, v, t4))
    j1 = jnp.where(b1, off, i1)
    j2 = jnp.where(b1, i1, jnp.where(b2, off, i2))
    j3 = jnp.where(b2, i2, jnp.where(b3, off, i3))
    j4 = jnp.where(b3, i3, jnp.where(b4, off, i4))
    return (n1, n2, n3, n4, j1, j2, j3, j4)


def _step(x_ref, sc_ref, eos_ref, ts_ref, wi_ref, bi_ref):
    sc = sc_ref[...]
    eosb = eos_ref[...] != 0

    lane = jax.lax.broadcasted_iota(jnp.int32, (_ROWS, _LANE), 1)
    row8 = jax.lax.broadcasted_iota(jnp.int32, (_ROWS, _LANE), 0)
    row4 = row8 % _BEAM

    v0 = x_ref[:, 0:_LANE]
    neg = jnp.full((_ROWS, _LANE), _NEG, jnp.float32)
    zero = jnp.zeros((_ROWS, _LANE), jnp.int32)
    empty = (neg, neg, neg, neg, zero, zero, zero, zero)
    states = [(jnp.where(lane < 2, _NEG, v0), neg, neg, neg,
               zero, zero, zero, zero)] + [empty] * (_PHASES - 1)

    for p in range(_PHASES):
        for c in range(1 + p, _NFULL, _PHASES):
            off = c * _LANE
            v = x_ref[:, off:off + _LANE]
            states[p] = _insert(states[p], v, off)

    vt = jnp.where(lane >= _TAILSKIP, x_ref[:, _TAILSTART:_TAILSTART + _LANE],
                   _NEG)
    states[0] = _insert(states[0], vt, _TAILSTART)

    mrow = jnp.maximum(jnp.maximum(states[0][0], states[1][0]),
                       jnp.maximum(states[2][0], states[3][0]))
    mrow = jnp.maximum(mrow, jnp.where(lane < 2, v0, _NEG))
    m = jnp.max(mrow, axis=1, keepdims=True)

    zerof = jnp.zeros((_ROWS, _LANE), jnp.float32)
    parts = [zerof, zerof, zerof, zerof]
    for c in range(_NFULL):
        off = c * _LANE
        parts[c % _PHASES] = parts[c % _PHASES] + \
            jnp.exp(x_ref[:, off:off + _LANE] - m)
    parts[0] = parts[0] + jnp.exp(vt - m)
    ssum = (parts[0] + parts[1]) + (parts[2] + parts[3])
    s = jnp.sum(ssum, axis=1, keepdims=True)
    crow = sc - m - jnp.log(s)
    crow = jnp.where(eosb, _NEG, crow)

    cv = []
    cf = []
    for st in states:
        t1, t2, t3, t4, i1, i2, i3, i4 = st
        cv += [t1 + crow, t2 + crow, t3 + crow, t4 + crow]
        cf += [row4 * _VOCAB + i1 + lane, row4 * _VOCAB + i2 + lane,
               row4 * _VOCAB + i3 + lane, row4 * _VOCAB + i4 + lane]
    cv.append(jnp.where(eosb & (lane == _EOS), sc, _NEG))
    cf.append(row4 * _VOCAB + lane)

    def _treemax(xs):
        while len(xs) > 1:
            xs = [jnp.maximum(a, b) for a, b in zip(xs[::2], xs[1::2])] + \
                ([xs[-1]] if len(xs) % 2 else [])
        return xs[0]

    def _treemin(xs):
        while len(xs) > 1:
            xs = [jnp.minimum(a, b) for a, b in zip(xs[::2], xs[1::2])] + \
                ([xs[-1]] if len(xs) % 2 else [])
        return xs[0]

    kcol = jax.lax.broadcasted_iota(jnp.int32, (1, 2, _BEAM), 2)
    brow = jax.lax.broadcasted_iota(jnp.int32, (1, 2, _BEAM), 1)
    ts = jnp.zeros((1, 2, _BEAM), jnp.float32)
    wi = jnp.zeros((1, 2, _BEAM), jnp.int32)
    bi = jnp.zeros((1, 2, _BEAM), jnp.int32)

    for b in range(2):
        selb = (row8 // _BEAM) == b
        av = [jnp.where(selb, v, _NEG) for v in cv]
        for k in range(_BEAM):
            mk = jnp.max(_treemax(av))
            idx = jnp.min(_treemin(
                [jnp.where(v == mk, f, _BIG) for v, f in zip(av, cf)]))
            av = [jnp.where((f == idx) & (v == mk), _NEG, v)
                  for v, f in zip(av, cf)]
            beam = idx // _VOCAB
            word = idx - beam * _VOCAB
            hit = (brow == b) & (kcol == k)
            ts = jnp.where(hit, mk, ts)
            wi = jnp.where(hit, word, wi)
            bi = jnp.where(hit, beam, bi)

    ts_ref[...] = ts
    wi_ref[...] = wi
    bi_ref[...] = bi


def kernel(logits, scores, generated_tokens, position):
    n_rows = logits.shape[0]
    n_batch = n_rows // _BEAM
    sc_sel = jnp.take(scores, position - 1, axis=2).reshape(n_rows, 1)
    eos = (jnp.take(generated_tokens, position, axis=1) == _EOS)
    eos = eos.astype(jnp.int32).reshape(n_rows, 1)

    grid = (n_rows // _ROWS,)
    out3 = (grid[0], 2, _BEAM)
    ts, wi, bi = pl.pallas_call(
        _step,
        grid=grid,
        in_specs=[
            pl.BlockSpec((_ROWS, _VOCAB), lambda i: (i, 0)),
            pl.BlockSpec((_ROWS, 1), lambda i: (i, 0)),
            pl.BlockSpec((_ROWS, 1), lambda i: (i, 0)),
        ],
        out_specs=[
            pl.BlockSpec((1, 2, _BEAM), lambda i: (i, 0, 0)),
            pl.BlockSpec((1, 2, _BEAM), lambda i: (i, 0, 0)),
            pl.BlockSpec((1, 2, _BEAM), lambda i: (i, 0, 0)),
        ],
        out_shape=[
            jax.ShapeDtypeStruct(out3, jnp.float32),
            jax.ShapeDtypeStruct(out3, jnp.int32),
            jax.ShapeDtypeStruct(out3, jnp.int32),
        ],
        compiler_params=pltpu.CompilerParams(
            dimension_semantics=("parallel",),
        ),
    )(logits, sc_sel, eos)
    return (ts.reshape(n_batch, _BEAM),
            wi.reshape(n_batch, _BEAM),
            bi.reshape(n_batch, _BEAM))
